# baseline (speedup 1.0000x reference)
import functools

import jax
import jax.numpy as jnp
from jax import lax
from jax.experimental import pallas as pl
from jax.experimental.pallas import tpu as pltpu
from jax.experimental.pallas import tpu_sc as plsc


_M = 512
_IC = 512


def _router_body(x_ref, g_ref, w_ref, e_ref):
    x = x_ref[...]
    logits = jnp.dot(x, g_ref[...], preferred_element_type=jnp.float32)
    t, e = logits.shape
    iota = jax.lax.broadcasted_iota(jnp.int32, (t, e), 1)
    m1 = jnp.max(logits, axis=1, keepdims=True)
    i1 = jnp.min(jnp.where(logits == m1, iota, e), axis=1, keepdims=True)
    masked = jnp.where(iota == i1, -jnp.inf, logits)
    m2 = jnp.max(masked, axis=1, keepdims=True)
    i2 = jnp.min(jnp.where(masked == m2, iota, e), axis=1, keepdims=True)
    wa = jax.nn.sigmoid(m1 - m2)
    w_ref[:, 0:1] = wa
    w_ref[:, 1:2] = 1.0 - wa
    e_ref[:, 0:1] = i1
    e_ref[:, 1:2] = i2


def _mlp_body(meta_ref, x_ref, rw_ref, w1_ref, w3_ref, w2_ref, out_ref):
    p = pl.program_id(0)
    ic = pl.program_id(1)
    lo = meta_ref[p, 2]
    hi = meta_ref[p, 3]
    init = meta_ref[p, 4]

    x = x_ref[...]
    g = jnp.dot(x, w1_ref[0], preferred_element_type=jnp.float32)
    u = jnp.dot(x, w3_ref[0], preferred_element_type=jnp.float32)
    h = (g * jax.nn.sigmoid(g)) * u
    rows = jax.lax.broadcasted_iota(jnp.int32, (x.shape[0], 1), 0)
    keep = (rows >= lo) & (rows < hi)
    h = jnp.where(keep, h * rw_ref[...], 0.0)
    contrib = jnp.dot(h, w2_ref[0], preferred_element_type=jnp.float32)

    first = jnp.logical_and(init == 1, ic == 0)

    @pl.when(first)
    def _():
        out_ref[...] = contrib

    @pl.when(jnp.logical_not(first))
    def _():
        out_ref[...] += contrib


def _sc_row_gather(table, idx):
    n_rows = idx.shape[0]
    d = table.shape[1]
    info = plsc.get_sparse_core_info()
    nw = info.num_cores * info.num_subcores
    b_per_w = n_rows // nw
    chunk = 16
    n_chunks = b_per_w // chunk

    @functools.partial(
        pl.kernel,
        mesh=plsc.VectorSubcoreMesh(core_axis_name="c", subcore_axis_name="s"),
        out_type=jax.ShapeDtypeStruct((n_rows, d), jnp.float32),
        scratch_types=[
            pltpu.VMEM((b_per_w,), jnp.int32),
            pltpu.VMEM((chunk, d), jnp.float32),
            pltpu.VMEM((chunk, d), jnp.float32),
            pltpu.SemaphoreType.DMA,
            pltpu.SemaphoreType.DMA,
        ],
    )
    def _gather(tab_hbm, idx_hbm, out_hbm, idx_v, r0_v, r1_v, sem0, sem1):
        wid = lax.axis_index("s") * info.num_cores + lax.axis_index("c")
        base = pl.multiple_of(wid * b_per_w, chunk)
        pltpu.sync_copy(idx_hbm.at[pl.ds(base, b_per_w)], idx_v)

        def chunk_body(c, carry):
            off0 = pl.multiple_of(c * 2 * chunk, chunk)
            off1 = pl.multiple_of(c * 2 * chunk + chunk, chunk)
            cp0 = pltpu.async_copy(tab_hbm.at[idx_v.at[pl.ds(off0, chunk)]],
                                   r0_v, sem0)
            cp1 = pltpu.async_copy(tab_hbm.at[idx_v.at[pl.ds(off1, chunk)]],
                                   r1_v, sem1)
            cp0.wait()
            pltpu.sync_copy(r0_v, out_hbm.at[pl.ds(
                pl.multiple_of(base + off0, chunk), chunk)])
            cp1.wait()
            pltpu.sync_copy(r1_v, out_hbm.at[pl.ds(
                pl.multiple_of(base + off1, chunk), chunk)])
            return carry

        lax.fori_loop(0, n_chunks // 2, chunk_body, 0)

    return _gather(table, idx)


def kernel(hidden_states, gate_w, w1s, w2s, w3s):
    t, h_dim = hidden_states.shape
    n_exp, _, i_dim = w1s.shape
    top_k = 2
    n = t * top_k
    m = _M
    num_m = n // m
    n_ic = i_dim // _IC
    n_pairs = num_m + n_exp - 1

    x = hidden_states.reshape(-1, h_dim)

    weights, experts = pl.pallas_call(
        _router_body,
        out_shape=[
            jax.ShapeDtypeStruct((t, top_k), jnp.float32),
            jax.ShapeDtypeStruct((t, top_k), jnp.int32),
        ],
    )(x, gate_w)

    flat_sel = experts.reshape(-1)
    order = jnp.argsort(flat_sel, stable=True)
    pos = jnp.argsort(order)
    token_idx = order // top_k
    x_sorted = _sc_row_gather(x, token_idx)
    rw_sorted = jnp.take(weights.reshape(-1), order, axis=0).reshape(n, 1)
    sizes = jnp.bincount(flat_sel, length=n_exp)
    offsets = jnp.concatenate([jnp.zeros((1,), jnp.int32),
                               jnp.cumsum(sizes).astype(jnp.int32)])
    tile_lo = (jnp.arange(num_m, dtype=jnp.int32) * m)[:, None]
    seg_lo = offsets[:-1][None, :]
    seg_hi = offsets[1:][None, :]
    ov_lo = jnp.maximum(seg_lo, tile_lo)
    ov_hi = jnp.minimum(seg_hi, tile_lo + m)
    active = ov_hi > ov_lo
    mm = jnp.broadcast_to(tile_lo // m, active.shape)
    ee = jnp.broadcast_to(jnp.arange(n_exp, dtype=jnp.int32)[None, :], active.shape)
    score = jnp.where(active, mm * n_exp + ee, num_m * n_exp + n_exp).reshape(-1)
    order64 = jnp.argsort(score)
    num_active = jnp.sum(active.astype(jnp.int32))
    sel = jnp.where(jnp.arange(n_pairs) < num_active,
                    order64[:n_pairs], order64[num_active - 1])
    m_p = (sel // n_exp).astype(jnp.int32)
    e_p = (sel % n_exp).astype(jnp.int32)
    lo_p = jnp.maximum(offsets[e_p], m_p * m) - m_p * m
    hi_p = jnp.minimum(offsets[e_p + 1], (m_p + 1) * m) - m_p * m
    hi_p = jnp.where(jnp.arange(n_pairs) < num_active, hi_p, lo_p)
    init_p = jnp.concatenate([jnp.ones((1,), jnp.int32),
                              (m_p[1:] != m_p[:-1]).astype(jnp.int32)])
    meta = jnp.stack([m_p, e_p, lo_p, hi_p, init_p], axis=1).astype(jnp.int32)

    grid_spec = pltpu.PrefetchScalarGridSpec(
        num_scalar_prefetch=1,
        grid=(n_pairs, n_ic),
        in_specs=[
            pl.BlockSpec((m, h_dim), lambda p, ic, md: (md[p, 0], 0)),
            pl.BlockSpec((m, 1), lambda p, ic, md: (md[p, 0], 0)),
            pl.BlockSpec((1, h_dim, _IC), lambda p, ic, md: (md[p, 1], 0, ic)),
            pl.BlockSpec((1, h_dim, _IC), lambda p, ic, md: (md[p, 1], 0, ic)),
            pl.BlockSpec((1, _IC, h_dim), lambda p, ic, md: (md[p, 1], ic, 0)),
        ],
        out_specs=pl.BlockSpec((m, h_dim), lambda p, ic, md: (md[p, 0], 0)),
    )
    sorted_out = pl.pallas_call(
        _mlp_body,
        grid_spec=grid_spec,
        out_shape=jax.ShapeDtypeStruct((n, h_dim), jnp.float32),
        compiler_params=pltpu.CompilerParams(
            dimension_semantics=("arbitrary", "arbitrary"),
        ),
    )(meta, x_sorted, rw_sorted, w1s, w3s, w2s)

    pos2 = pos.reshape(t, top_k)
    i0 = pos2[:, 0]
    i1 = pos2[:, 1]

    info = plsc.get_sparse_core_info()
    nw = info.num_cores * info.num_subcores
    b_per_w = t // nw
    chunk = 8
    n_chunks = b_per_w // chunk

    @functools.partial(
        pl.kernel,
        mesh=plsc.VectorSubcoreMesh(core_axis_name="c", subcore_axis_name="s"),
        out_type=jax.ShapeDtypeStruct((t, h_dim), jnp.float32),
        scratch_types=[
            pltpu.VMEM((b_per_w,), jnp.int32),
            pltpu.VMEM((b_per_w,), jnp.int32),
            pltpu.VMEM((chunk, h_dim), jnp.float32),
            pltpu.VMEM((chunk, h_dim), jnp.float32),
            pltpu.VMEM((chunk, h_dim), jnp.float32),
            pltpu.VMEM((chunk, h_dim), jnp.float32),
            pltpu.SemaphoreType.DMA,
            pltpu.SemaphoreType.DMA,
            pltpu.SemaphoreType.DMA,
            pltpu.SemaphoreType.DMA,
        ],
    )
    def _combine(y_hbm, i0_hbm, i1_hbm, out_hbm,
                 idx0_v, idx1_v, r0a_v, r1a_v, r0b_v, r1b_v,
                 sem0a, sem1a, sem0b, sem1b):
        wid = lax.axis_index("s") * info.num_cores + lax.axis_index("c")
        base = pl.multiple_of(wid * b_per_w, chunk)
        pltpu.sync_copy(i0_hbm.at[pl.ds(base, b_per_w)], idx0_v)
        pltpu.sync_copy(i1_hbm.at[pl.ds(base, b_per_w)], idx1_v)

        def start(c, r0_v, r1_v, s0, s1):
            off = pl.multiple_of(
                jnp.minimum(c, n_chunks - 1) * chunk, chunk)
            cp0 = pltpu.async_copy(y_hbm.at[idx0_v.at[pl.ds(off, chunk)]],
                                   r0_v, s0)
            cp1 = pltpu.async_copy(y_hbm.at[idx1_v.at[pl.ds(off, chunk)]],
                                   r1_v, s1)
            return cp0, cp1

        def finish(c, r0_v, r1_v, cp0, cp1):
            cp0.wait()
            cp1.wait()

            def add_body(k, carry2):
                i = k // (h_dim // 256)
                j = (k % (h_dim // 256)) * 256
                for u in range(16):
                    sl = pl.ds(pl.multiple_of(j + u * 16, 16), 16)
                    r0_v[i, sl] = r0_v[i, sl] + r1_v[i, sl]
                return carry2

            lax.fori_loop(0, chunk * (h_dim // 256), add_body, 0)
            off = pl.multiple_of(c * chunk, chunk)
            pltpu.sync_copy(
                r0_v, out_hbm.at[pl.ds(pl.multiple_of(base + off, chunk), chunk)])

        cpa = start(0, r0a_v, r1a_v, sem0a, sem1a)

        def pair_body(c2, carry):
            ca = c2 * 2
            cpb = start(ca + 1, r0b_v, r1b_v, sem0b, sem1b)
            finish(ca, r0a_v, r1a_v,
                   pltpu.make_async_copy(y_hbm.at[idx0_v.at[pl.ds(0, chunk)]],
                                         r0a_v, sem0a),
                   pltpu.make_async_copy(y_hbm.at[idx1_v.at[pl.ds(0, chunk)]],
                                         r1a_v, sem1a))
            cpa2 = start(ca + 2, r0a_v, r1a_v, sem0a, sem1a)
            finish(ca + 1, r0b_v, r1b_v, cpb[0], cpb[1])
            return carry

        lax.fori_loop(0, n_chunks // 2, pair_body, 0)
        pltpu.make_async_copy(y_hbm.at[idx0_v.at[pl.ds(0, chunk)]],
                              r0a_v, sem0a).wait()
        pltpu.make_async_copy(y_hbm.at[idx1_v.at[pl.ds(0, chunk)]],
                              r1a_v, sem1a).wait()

    return _combine(sorted_out, i0, i1)


# speedup vs baseline: 1.0136x; 1.0136x over previous
import functools

import jax
import jax.numpy as jnp
from jax import lax
from jax.experimental import pallas as pl
from jax.experimental.pallas import tpu as pltpu
from jax.experimental.pallas import tpu_sc as plsc


_M = 512
_IC = 512


def _router_body(x_ref, g_ref, w_ref, e_ref):
    x = x_ref[...]
    logits = jnp.dot(x, g_ref[...], preferred_element_type=jnp.float32)
    t, e = logits.shape
    iota = jax.lax.broadcasted_iota(jnp.int32, (t, e), 1)
    m1 = jnp.max(logits, axis=1, keepdims=True)
    i1 = jnp.min(jnp.where(logits == m1, iota, e), axis=1, keepdims=True)
    masked = jnp.where(iota == i1, -jnp.inf, logits)
    m2 = jnp.max(masked, axis=1, keepdims=True)
    i2 = jnp.min(jnp.where(masked == m2, iota, e), axis=1, keepdims=True)
    wa = jax.nn.sigmoid(m1 - m2)
    w_ref[:, 0:1] = wa
    w_ref[:, 1:2] = 1.0 - wa
    e_ref[:, 0:1] = i1
    e_ref[:, 1:2] = i2


def _mlp_body(meta_ref, x_ref, rw_ref, w1_ref, w3_ref, w2_ref, out_ref):
    p = pl.program_id(0)
    ic = pl.program_id(1)
    lo = meta_ref[p, 2]
    hi = meta_ref[p, 3]
    init = meta_ref[p, 4]

    x = x_ref[...]
    g = jnp.dot(x, w1_ref[0], preferred_element_type=jnp.float32)
    u = jnp.dot(x, w3_ref[0], preferred_element_type=jnp.float32)
    h = (g * jax.nn.sigmoid(g)) * u
    rows = jax.lax.broadcasted_iota(jnp.int32, (x.shape[0], 1), 0)
    keep = (rows >= lo) & (rows < hi)
    h = jnp.where(keep, h * rw_ref[...], 0.0)
    contrib = jnp.dot(h, w2_ref[0], preferred_element_type=jnp.float32)

    first = jnp.logical_and(init == 1, ic == 0)

    @pl.when(first)
    def _():
        out_ref[...] = contrib

    @pl.when(jnp.logical_not(first))
    def _():
        out_ref[...] += contrib


def _sc_row_gather(table, idx):
    n_rows = idx.shape[0]
    d = table.shape[1]
    info = plsc.get_sparse_core_info()
    nw = info.num_cores * info.num_subcores
    b_per_w = n_rows // nw
    chunk = 16
    n_chunks = b_per_w // chunk

    @functools.partial(
        pl.kernel,
        mesh=plsc.VectorSubcoreMesh(core_axis_name="c", subcore_axis_name="s"),
        out_type=jax.ShapeDtypeStruct((n_rows, d), jnp.float32),
        scratch_types=[
            pltpu.VMEM((b_per_w,), jnp.int32),
            pltpu.VMEM((chunk, d), jnp.float32),
            pltpu.VMEM((chunk, d), jnp.float32),
            pltpu.SemaphoreType.DMA,
            pltpu.SemaphoreType.DMA,
        ],
    )
    def _gather(tab_hbm, idx_hbm, out_hbm, idx_v, r0_v, r1_v, sem0, sem1):
        wid = lax.axis_index("s") * info.num_cores + lax.axis_index("c")
        base = pl.multiple_of(wid * b_per_w, chunk)
        pltpu.sync_copy(idx_hbm.at[pl.ds(base, b_per_w)], idx_v)

        def chunk_body(c, carry):
            off0 = pl.multiple_of(c * 2 * chunk, chunk)
            off1 = pl.multiple_of(c * 2 * chunk + chunk, chunk)
            cp0 = pltpu.async_copy(tab_hbm.at[idx_v.at[pl.ds(off0, chunk)]],
                                   r0_v, sem0)
            cp1 = pltpu.async_copy(tab_hbm.at[idx_v.at[pl.ds(off1, chunk)]],
                                   r1_v, sem1)
            cp0.wait()
            pltpu.sync_copy(r0_v, out_hbm.at[pl.ds(
                pl.multiple_of(base + off0, chunk), chunk)])
            cp1.wait()
            pltpu.sync_copy(r1_v, out_hbm.at[pl.ds(
                pl.multiple_of(base + off1, chunk), chunk)])
            return carry

        lax.fori_loop(0, n_chunks // 2, chunk_body, 0)

    return _gather(table, idx)


def kernel(hidden_states, gate_w, w1s, w2s, w3s):
    t, h_dim = hidden_states.shape
    n_exp, _, i_dim = w1s.shape
    top_k = 2
    n = t * top_k
    m = _M
    num_m = n // m
    n_ic = i_dim // _IC
    n_pairs = num_m + n_exp - 1

    x = hidden_states.reshape(-1, h_dim)

    weights, experts = pl.pallas_call(
        _router_body,
        out_shape=[
            jax.ShapeDtypeStruct((t, top_k), jnp.float32),
            jax.ShapeDtypeStruct((t, top_k), jnp.int32),
        ],
    )(x, gate_w)

    flat_sel = experts.reshape(-1)
    order = jnp.argsort(flat_sel, stable=True)
    pos = jnp.argsort(order)
    token_idx = order // top_k
    x_sorted = _sc_row_gather(x, token_idx)
    rw_sorted = jnp.take(weights.reshape(-1), order, axis=0).reshape(n, 1)
    sizes = jnp.bincount(flat_sel, length=n_exp)
    offsets = jnp.concatenate([jnp.zeros((1,), jnp.int32),
                               jnp.cumsum(sizes).astype(jnp.int32)])
    tile_lo = (jnp.arange(num_m, dtype=jnp.int32) * m)[:, None]
    seg_lo = offsets[:-1][None, :]
    seg_hi = offsets[1:][None, :]
    ov_lo = jnp.maximum(seg_lo, tile_lo)
    ov_hi = jnp.minimum(seg_hi, tile_lo + m)
    active = ov_hi > ov_lo
    mm = jnp.broadcast_to(tile_lo // m, active.shape)
    ee = jnp.broadcast_to(jnp.arange(n_exp, dtype=jnp.int32)[None, :], active.shape)
    score = jnp.where(active, mm * n_exp + ee, num_m * n_exp + n_exp).reshape(-1)
    order64 = jnp.argsort(score)
    num_active = jnp.sum(active.astype(jnp.int32))
    sel = jnp.where(jnp.arange(n_pairs) < num_active,
                    order64[:n_pairs], order64[num_active - 1])
    m_p = (sel // n_exp).astype(jnp.int32)
    e_p = (sel % n_exp).astype(jnp.int32)
    lo_p = jnp.maximum(offsets[e_p], m_p * m) - m_p * m
    hi_p = jnp.minimum(offsets[e_p + 1], (m_p + 1) * m) - m_p * m
    hi_p = jnp.where(jnp.arange(n_pairs) < num_active, hi_p, lo_p)
    init_p = jnp.concatenate([jnp.ones((1,), jnp.int32),
                              (m_p[1:] != m_p[:-1]).astype(jnp.int32)])
    meta = jnp.stack([m_p, e_p, lo_p, hi_p, init_p], axis=1).astype(jnp.int32)

    grid_spec = pltpu.PrefetchScalarGridSpec(
        num_scalar_prefetch=1,
        grid=(n_pairs, n_ic),
        in_specs=[
            pl.BlockSpec((m, h_dim), lambda p, ic, md: (md[p, 0], 0)),
            pl.BlockSpec((m, 1), lambda p, ic, md: (md[p, 0], 0)),
            pl.BlockSpec((1, h_dim, _IC), lambda p, ic, md: (md[p, 1], 0, ic)),
            pl.BlockSpec((1, h_dim, _IC), lambda p, ic, md: (md[p, 1], 0, ic)),
            pl.BlockSpec((1, _IC, h_dim), lambda p, ic, md: (md[p, 1], ic, 0)),
        ],
        out_specs=pl.BlockSpec((m, h_dim), lambda p, ic, md: (md[p, 0], 0)),
    )
    sorted_out = pl.pallas_call(
        _mlp_body,
        grid_spec=grid_spec,
        out_shape=jax.ShapeDtypeStruct((n, h_dim), jnp.float32),
        compiler_params=pltpu.CompilerParams(
            dimension_semantics=("arbitrary", "arbitrary"),
        ),
    )(meta, x_sorted, rw_sorted, w1s, w3s, w2s)

    pos2 = pos.reshape(t, top_k)
    i0 = pos2[:, 0]
    i1 = pos2[:, 1]

    info = plsc.get_sparse_core_info()
    nw = info.num_cores * info.num_subcores
    b_per_w = t // nw
    chunk = 8
    n_chunks = b_per_w // chunk

    @functools.partial(
        pl.kernel,
        mesh=plsc.VectorSubcoreMesh(core_axis_name="c", subcore_axis_name="s"),
        out_type=jax.ShapeDtypeStruct((t, h_dim), jnp.float32),
        scratch_types=[
            pltpu.VMEM((b_per_w,), jnp.int32),
            pltpu.VMEM((b_per_w,), jnp.int32),
            pltpu.VMEM((chunk, h_dim), jnp.float32),
            pltpu.VMEM((chunk, h_dim), jnp.float32),
            pltpu.VMEM((chunk, h_dim), jnp.float32),
            pltpu.VMEM((chunk, h_dim), jnp.float32),
            pltpu.SemaphoreType.DMA,
            pltpu.SemaphoreType.DMA,
            pltpu.SemaphoreType.DMA,
            pltpu.SemaphoreType.DMA,
        ],
    )
    def _combine(y_hbm, i0_hbm, i1_hbm, out_hbm,
                 idx0_v, idx1_v, r0a_v, r1a_v, r0b_v, r1b_v,
                 sem0a, sem1a, sem0b, sem1b):
        wid = lax.axis_index("s") * info.num_cores + lax.axis_index("c")
        base = pl.multiple_of(wid * b_per_w, chunk)
        pltpu.sync_copy(i0_hbm.at[pl.ds(base, b_per_w)], idx0_v)
        pltpu.sync_copy(i1_hbm.at[pl.ds(base, b_per_w)], idx1_v)

        def start(c, r0_v, r1_v, s0, s1):
            off = pl.multiple_of(
                jnp.minimum(c, n_chunks - 1) * chunk, chunk)
            cp0 = pltpu.async_copy(y_hbm.at[idx0_v.at[pl.ds(off, chunk)]],
                                   r0_v, s0)
            cp1 = pltpu.async_copy(y_hbm.at[idx1_v.at[pl.ds(off, chunk)]],
                                   r1_v, s1)
            return cp0, cp1

        def finish(c, r0_v, r1_v, cp0, cp1):
            cp0.wait()
            cp1.wait()

            def add_body(k, carry2):
                i = k // (h_dim // 64)
                j = (k % (h_dim // 64)) * 64
                for u in range(4):
                    sl = pl.ds(pl.multiple_of(j + u * 16, 16), 16)
                    r0_v[i, sl] = r0_v[i, sl] + r1_v[i, sl]
                return carry2

            lax.fori_loop(0, chunk * (h_dim // 64), add_body, 0)
            off = pl.multiple_of(c * chunk, chunk)
            pltpu.sync_copy(
                r0_v, out_hbm.at[pl.ds(pl.multiple_of(base + off, chunk), chunk)])

        cpa = start(0, r0a_v, r1a_v, sem0a, sem1a)

        def pair_body(c2, carry):
            ca = c2 * 2
            cpb = start(ca + 1, r0b_v, r1b_v, sem0b, sem1b)
            finish(ca, r0a_v, r1a_v,
                   pltpu.make_async_copy(y_hbm.at[idx0_v.at[pl.ds(0, chunk)]],
                                         r0a_v, sem0a),
                   pltpu.make_async_copy(y_hbm.at[idx1_v.at[pl.ds(0, chunk)]],
                                         r1a_v, sem1a))
            cpa2 = start(ca + 2, r0a_v, r1a_v, sem0a, sem1a)
            finish(ca + 1, r0b_v, r1b_v, cpb[0], cpb[1])
            return carry

        lax.fori_loop(0, n_chunks // 2, pair_body, 0)
        pltpu.make_async_copy(y_hbm.at[idx0_v.at[pl.ds(0, chunk)]],
                              r0a_v, sem0a).wait()
        pltpu.make_async_copy(y_hbm.at[idx1_v.at[pl.ds(0, chunk)]],
                              r1a_v, sem1a).wait()

    return _combine(sorted_out, i0, i1)
